# Initial kernel scaffold; baseline (speedup 1.0000x reference)
import jax
import jax.numpy as jnp
from jax import lax
from jax.experimental import pallas as pl
from jax.experimental.pallas import tpu as pltpu
from jax.experimental.pallas import tpu_sc as plsc

_PENALTY = 1.2
_B, _V, _T = 64, 100000, 2048
_L = 16
_NW = 32
_ROWS_PER_W = _B // _NW


def _body(ids_hbm, scores_hbm, out_hbm, row_v, idx_v, val_v):
    c = lax.axis_index("c")
    s = lax.axis_index("s")
    wid = s * 2 + c

    for r in range(_ROWS_PER_W):
        row = wid * _ROWS_PER_W + r
        pltpu.sync_copy(scores_hbm.at[row], row_v)
        pltpu.sync_copy(ids_hbm.at[row], idx_v)

        def gather_body(i, _):
            idx = idx_v[pl.ds(i * _L, _L)]
            vals = plsc.load_gather(row_v, [idx])
            pen = jnp.where(vals < 0.0, vals * _PENALTY, vals / _PENALTY)
            val_v[pl.ds(i * _L, _L)] = pen
            return 0

        lax.fori_loop(0, _T // _L, gather_body, 0)

        def scatter_body(i, _):
            idx = idx_v[pl.ds(i * _L, _L)]
            plsc.store_scatter(row_v, [idx], val_v[pl.ds(i * _L, _L)])
            return 0

        lax.fori_loop(0, _T // _L, scatter_body, 0)

        pltpu.sync_copy(row_v, out_hbm.at[row])


@jax.jit
def _run(input_ids, scores):
    mesh = plsc.VectorSubcoreMesh(core_axis_name="c", subcore_axis_name="s")
    return pl.kernel(
        _body,
        mesh=mesh,
        out_type=jax.ShapeDtypeStruct((_B, _V), jnp.float32),
        scratch_types=[
            pltpu.VMEM((_V,), jnp.float32),
            pltpu.VMEM((_T,), jnp.int32),
            pltpu.VMEM((_T,), jnp.float32),
        ],
    )(input_ids, scores)


def kernel(input_ids, scores):
    return _run(input_ids.astype(jnp.int32), scores)


# speedup vs baseline: 14.2764x; 14.2764x over previous
import jax
import jax.numpy as jnp
from jax import lax
from jax.experimental import pallas as pl
from jax.experimental.pallas import tpu as pltpu
from jax.experimental.pallas import tpu_sc as plsc

_PENALTY = 1.2
_B, _V, _T = 64, 100000, 2048
_L = 16
_NW = 32
_ROWS_PER_W = _B // _NW


def _body(ids_hbm, scores_hbm, out_hbm, row_v, idx_v, val_v):
    c = lax.axis_index("c")
    s = lax.axis_index("s")
    wid = s * 2 + c

    for r in range(_ROWS_PER_W):
        row = wid * _ROWS_PER_W + r
        pltpu.sync_copy(scores_hbm.at[row], row_v)
        pltpu.sync_copy(ids_hbm.at[row], idx_v)

        def gather_body(i, _):
            idx = idx_v[pl.ds(i * _L, _L)]
            vals = plsc.load_gather(row_v, [idx])
            pen = jnp.where(vals < 0.0, vals * _PENALTY, vals / _PENALTY)
            val_v[pl.ds(i * _L, _L)] = pen
            return 0

        lax.fori_loop(0, _T // _L, gather_body, 0)

        def scatter_body(i, _):
            idx = idx_v[pl.ds(i * _L, _L)]
            plsc.store_scatter(row_v, [idx], val_v[pl.ds(i * _L, _L)])
            return 0

        lax.fori_loop(0, _T // _L, scatter_body, 0)

        pltpu.sync_copy(row_v, out_hbm.at[row])


@jax.jit
def _run(input_ids, scores):
    mesh = plsc.VectorSubcoreMesh(core_axis_name="c", subcore_axis_name="s")
    return pl.kernel(
        _body,
        mesh=mesh,
        out_type=jax.ShapeDtypeStruct((_B, _V), jnp.float32),
        scratch_types=[
            pltpu.VMEM((_V,), jnp.float32),
            pltpu.VMEM((_T,), jnp.int32),
            pltpu.VMEM((_T,), jnp.float32),
        ],
        compiler_params=pltpu.CompilerParams(needs_layout_passes=False),
    )(input_ids, scores)


def kernel(input_ids, scores):
    return _run(input_ids.astype(jnp.int32), scores)
